# baseline (speedup 1.0000x reference)
import functools

import jax
import jax.numpy as jnp
from jax import lax
from jax.experimental import pallas as pl
from jax.experimental.pallas import tpu as pltpu
from jax.experimental.pallas import tpu_sc as plsc

N_ROWS = 64
N_COLS = 32768
NUM_CORES = 2
NUM_SUBCORES = 16
NUM_WORKERS = NUM_CORES * NUM_SUBCORES
ROWS_PER_WORKER = N_ROWS // NUM_WORKERS
LANES = 16
UNROLL = 8
SEG = 4096
SEGS_PER_ROW = N_COLS // SEG
SEGS_PER_WORKER = ROWS_PER_WORKER * SEGS_PER_ROW
ITERS_PER_SEG = SEG // (UNROLL * LANES)
ITERS_PER_ROW = SEGS_PER_ROW * ITERS_PER_SEG

_mesh = plsc.VectorSubcoreMesh(core_axis_name="c", subcore_axis_name="s")


def _lex_min(av, ai, bv, bi):
    upd = (bv < av) | ((bv == av) & (bi < ai))
    return jnp.where(upd, bv, av), jnp.where(upd, bi, ai)


@functools.partial(
    pl.kernel,
    mesh=_mesh,
    out_type=jax.ShapeDtypeStruct((NUM_WORKERS, LANES), jnp.int32),
    scratch_types=[
        pltpu.VMEM((SEG,), jnp.float32),
        pltpu.VMEM((SEG,), jnp.float32),
        pltpu.VMEM((LANES,), jnp.int32),
        pltpu.SemaphoreType.DMA,
        pltpu.SemaphoreType.DMA,
    ],
)
def _argmin_sc(x_hbm, out_hbm, buf0, buf1, res_v, sem0, sem1):
    wid = lax.axis_index("s") * NUM_CORES + lax.axis_index("c")
    base = wid * (ROWS_PER_WORKER * N_COLS)
    bufs = (buf0, buf1)
    sems = (sem0, sem1)

    def issue(g):
        b = g % 2
        return pltpu.async_copy(
            x_hbm.at[pl.ds(base + g * SEG, SEG)], bufs[b], sems[b])

    handles = [issue(0), issue(1)]

    lane_iota = lax.iota(jnp.int32, LANES)
    res_vec = jnp.zeros((LANES,), jnp.int32)

    for r in range(ROWS_PER_WORKER):
        bvs = [jnp.full((LANES,), jnp.inf, jnp.float32) for _ in range(UNROLL)]
        bis = [jnp.zeros((LANES,), jnp.int32) for _ in range(UNROLL)]

        for si in range(SEGS_PER_ROW):
            g = r * SEGS_PER_ROW + si
            b = g % 2
            handles[b].wait()
            buf = bufs[b]

            def seg_body(it, carry, buf=buf, si=si):
                acc = list(carry)
                git = si * ITERS_PER_SEG + it
                git_vec = jnp.full((LANES,), git, jnp.int32)
                off = it * (UNROLL * LANES)
                for u in range(UNROLL):
                    bv, bi = acc[2 * u], acc[2 * u + 1]
                    v = buf[pl.ds(off + u * LANES, LANES)]
                    upd = v < bv
                    acc[2 * u] = jnp.where(upd, v, bv)
                    acc[2 * u + 1] = jnp.where(upd, git_vec, bi)
                return tuple(acc)

            carry = []
            for u in range(UNROLL):
                carry += [bvs[u], bis[u]]
            carry = lax.fori_loop(0, ITERS_PER_SEG, seg_body, tuple(carry))
            bvs = [carry[2 * u] for u in range(UNROLL)]
            bis = [carry[2 * u + 1] for u in range(UNROLL)]

            ng = g + 2
            if ng < SEGS_PER_WORKER:
                handles[b] = issue(ng)

        mv, mi = None, None
        for u in range(UNROLL):
            gidx = bis[u] * (UNROLL * LANES) + (lane_iota + u * LANES)
            if mv is None:
                mv, mi = bvs[u], gidx
            else:
                mv, mi = _lex_min(mv, mi, bvs[u], gidx)

        for shift in (8, 4, 2, 1):
            partner = lane_iota ^ shift
            pv = mv.at[partner].get(mode="promise_in_bounds", unique_indices=True)
            pi = mi.at[partner].get(mode="promise_in_bounds", unique_indices=True)
            mv, mi = _lex_min(mv, mi, pv, pi)

        res_vec = jnp.where(lane_iota == r, mi, res_vec)

    res_v[...] = res_vec
    pltpu.sync_copy(res_v, out_hbm.at[wid])


def kernel(x):
    out2d = _argmin_sc(x.reshape(-1))
    return out2d[:, :ROWS_PER_WORKER].reshape(-1)


# speedup vs baseline: 1.3742x; 1.3742x over previous
import functools

import jax
import jax.numpy as jnp
from jax import lax
from jax.experimental import pallas as pl
from jax.experimental.pallas import tpu as pltpu
from jax.experimental.pallas import tpu_sc as plsc

N_ROWS = 64
N_COLS = 32768
NUM_CORES = 2
NUM_SUBCORES = 16
NUM_WORKERS = NUM_CORES * NUM_SUBCORES
ROWS_PER_WORKER = N_ROWS // NUM_WORKERS
LANES = 16
UNROLL = 8
SEG = 4096
SEGS_PER_ROW = N_COLS // SEG
SEGS_PER_WORKER = ROWS_PER_WORKER * SEGS_PER_ROW
ITERS_PER_SEG = SEG // (UNROLL * LANES)
ITERS_PER_ROW = SEGS_PER_ROW * ITERS_PER_SEG

_mesh = plsc.VectorSubcoreMesh(core_axis_name="c", subcore_axis_name="s")


def _lex_min(av, ai, bv, bi):
    upd = (bv < av) | ((bv == av) & (bi < ai))
    return jnp.where(upd, bv, av), jnp.where(upd, bi, ai)


@functools.partial(
    pl.kernel,
    mesh=_mesh,
    out_type=jax.ShapeDtypeStruct((NUM_WORKERS, LANES), jnp.int32),
    scratch_types=[
        pltpu.VMEM((SEG,), jnp.float32),
        pltpu.VMEM((SEG,), jnp.float32),
        pltpu.VMEM((LANES,), jnp.int32),
        pltpu.SemaphoreType.DMA,
        pltpu.SemaphoreType.DMA,
    ],
)
def _argmin_sc(x_hbm, out_hbm, buf0, buf1, res_v, sem0, sem1):
    wid = lax.axis_index("s") * NUM_CORES + lax.axis_index("c")
    base_row = wid * ROWS_PER_WORKER
    bufs = (buf0, buf1)
    sems = (sem0, sem1)

    def issue(g):
        b = g % 2
        r, si = divmod(g, SEGS_PER_ROW)
        return pltpu.async_copy(
            x_hbm.at[base_row + r, pl.ds(si * SEG, SEG)], bufs[b], sems[b])

    handles = [issue(0), issue(1)]

    lane_iota = lax.iota(jnp.int32, LANES)
    res_vec = jnp.zeros((LANES,), jnp.int32)

    for r in range(ROWS_PER_WORKER):
        bvs = [jnp.full((LANES,), jnp.inf, jnp.float32) for _ in range(UNROLL)]
        bis = [jnp.zeros((LANES,), jnp.int32) for _ in range(UNROLL)]

        for si in range(SEGS_PER_ROW):
            g = r * SEGS_PER_ROW + si
            b = g % 2
            handles[b].wait()
            buf = bufs[b]

            def seg_body(it, carry, buf=buf, si=si):
                acc = list(carry)
                git = si * ITERS_PER_SEG + it
                git_vec = jnp.full((LANES,), git, jnp.int32)
                off = it * (UNROLL * LANES)
                for u in range(UNROLL):
                    bv, bi = acc[2 * u], acc[2 * u + 1]
                    v = buf[pl.ds(off + u * LANES, LANES)]
                    upd = v < bv
                    acc[2 * u] = jnp.where(upd, v, bv)
                    acc[2 * u + 1] = jnp.where(upd, git_vec, bi)
                return tuple(acc)

            carry = []
            for u in range(UNROLL):
                carry += [bvs[u], bis[u]]
            carry = lax.fori_loop(0, ITERS_PER_SEG, seg_body, tuple(carry))
            bvs = [carry[2 * u] for u in range(UNROLL)]
            bis = [carry[2 * u + 1] for u in range(UNROLL)]

            ng = g + 2
            if ng < SEGS_PER_WORKER:
                handles[b] = issue(ng)

        mv, mi = None, None
        for u in range(UNROLL):
            gidx = bis[u] * (UNROLL * LANES) + (lane_iota + u * LANES)
            if mv is None:
                mv, mi = bvs[u], gidx
            else:
                mv, mi = _lex_min(mv, mi, bvs[u], gidx)

        for shift in (8, 4, 2, 1):
            partner = lane_iota ^ shift
            pv = mv.at[partner].get(mode="promise_in_bounds", unique_indices=True)
            pi = mi.at[partner].get(mode="promise_in_bounds", unique_indices=True)
            mv, mi = _lex_min(mv, mi, pv, pi)

        res_vec = jnp.where(lane_iota == r, mi, res_vec)

    res_v[...] = res_vec
    pltpu.sync_copy(res_v, out_hbm.at[wid])


def kernel(x):
    out2d = _argmin_sc(x)
    return out2d[:, :ROWS_PER_WORKER].reshape(-1)


# speedup vs baseline: 1.5381x; 1.1193x over previous
import functools

import jax
import jax.numpy as jnp
from jax import lax
from jax.experimental import pallas as pl
from jax.experimental.pallas import tpu as pltpu
from jax.experimental.pallas import tpu_sc as plsc

N_ROWS = 64
N_COLS = 32768
NUM_CORES = 2
NUM_SUBCORES = 16
NUM_WORKERS = NUM_CORES * NUM_SUBCORES
ROWS_PER_WORKER = N_ROWS // NUM_WORKERS
LANES = 16
UNROLL = 8
ITERS_PER_ROW = N_COLS // (UNROLL * LANES)

_mesh = plsc.VectorSubcoreMesh(core_axis_name="c", subcore_axis_name="s")


def _lex_min(av, ai, bv, bi):
    upd = (bv < av) | ((bv == av) & (bi < ai))
    return jnp.where(upd, bv, av), jnp.where(upd, bi, ai)


@functools.partial(
    pl.kernel,
    mesh=_mesh,
    out_type=jax.ShapeDtypeStruct((NUM_WORKERS, LANES), jnp.int32),
    scratch_types=[
        pltpu.VMEM((N_COLS,), jnp.float32),
        pltpu.VMEM((N_COLS,), jnp.float32),
        pltpu.VMEM((LANES,), jnp.int32),
        pltpu.SemaphoreType.DMA,
        pltpu.SemaphoreType.DMA,
    ],
)
def _argmin_sc(x_hbm, out_hbm, buf0, buf1, res_v, sem0, sem1):
    wid = lax.axis_index("s") * NUM_CORES + lax.axis_index("c")
    base_row = wid * ROWS_PER_WORKER

    handles = (
        pltpu.async_copy(x_hbm.at[base_row], buf0, sem0),
        pltpu.async_copy(x_hbm.at[base_row + 1], buf1, sem1),
    )
    bufs = (buf0, buf1)

    lane_iota = lax.iota(jnp.int32, LANES)
    res_vec = jnp.zeros((LANES,), jnp.int32)

    for r in range(ROWS_PER_WORKER):
        handles[r].wait()
        buf = bufs[r]

        def row_body(it, carry, buf=buf):
            acc = list(carry)
            git_vec = jnp.full((LANES,), it, jnp.int32)
            off = it * (UNROLL * LANES)
            for u in range(UNROLL):
                bv, bi = acc[2 * u], acc[2 * u + 1]
                v = buf[pl.ds(off + u * LANES, LANES)]
                upd = v < bv
                acc[2 * u] = jnp.where(upd, v, bv)
                acc[2 * u + 1] = jnp.where(upd, git_vec, bi)
            return tuple(acc)

        init = []
        for _ in range(UNROLL):
            init += [jnp.full((LANES,), jnp.inf, jnp.float32),
                     jnp.zeros((LANES,), jnp.int32)]
        carry = lax.fori_loop(0, ITERS_PER_ROW, row_body, tuple(init))

        mv, mi = None, None
        for u in range(UNROLL):
            gidx = carry[2 * u + 1] * (UNROLL * LANES) + (lane_iota + u * LANES)
            if mv is None:
                mv, mi = carry[2 * u], gidx
            else:
                mv, mi = _lex_min(mv, mi, carry[2 * u], gidx)

        for shift in (8, 4, 2, 1):
            partner = lane_iota ^ shift
            pv = mv.at[partner].get(mode="promise_in_bounds", unique_indices=True)
            pi = mi.at[partner].get(mode="promise_in_bounds", unique_indices=True)
            mv, mi = _lex_min(mv, mi, pv, pi)

        res_vec = jnp.where(lane_iota == r, mi, res_vec)

    res_v[...] = res_vec
    pltpu.sync_copy(res_v, out_hbm.at[wid])


def kernel(x):
    out2d = _argmin_sc(x)
    return out2d[:, :ROWS_PER_WORKER].reshape(-1)


# speedup vs baseline: 1.6058x; 1.0440x over previous
import functools

import jax
import jax.numpy as jnp
from jax import lax
from jax.experimental import pallas as pl
from jax.experimental.pallas import tpu as pltpu
from jax.experimental.pallas import tpu_sc as plsc

N_ROWS = 64
N_COLS = 32768
NUM_CORES = 2
NUM_SUBCORES = 16
NUM_WORKERS = NUM_CORES * NUM_SUBCORES
ROWS_PER_WORKER = N_ROWS // NUM_WORKERS
LANES = 16
UNROLL = 8
ITERS_PER_ROW = N_COLS // (UNROLL * LANES)

_mesh = plsc.VectorSubcoreMesh(core_axis_name="c", subcore_axis_name="s")


def _lex_min(av, ai, bv, bi):
    upd = (bv < av) | ((bv == av) & (bi < ai))
    return jnp.where(upd, bv, av), jnp.where(upd, bi, ai)


@functools.partial(
    pl.kernel,
    mesh=_mesh,
    out_type=jax.ShapeDtypeStruct((NUM_WORKERS, LANES), jnp.int32),
    scratch_types=[
        pltpu.VMEM((ROWS_PER_WORKER * N_COLS,), jnp.float32),
        pltpu.VMEM((LANES,), jnp.int32),
        pltpu.SemaphoreType.DMA,
    ],
)
def _argmin_sc(x_hbm, out_hbm, buf, res_v, sem):
    wid = lax.axis_index("s") * NUM_CORES + lax.axis_index("c")
    base_row = wid * ROWS_PER_WORKER

    for r in range(ROWS_PER_WORKER):
        pltpu.async_copy(x_hbm.at[base_row + r],
                         buf.at[pl.ds(r * N_COLS, N_COLS)], sem)

    lane_iota = lax.iota(jnp.int32, LANES)

    def per_row(r, res_vec):
        pltpu.make_async_copy(
            x_hbm.at[base_row], buf.at[pl.ds(0, N_COLS)], sem).wait()
        row_off = r * N_COLS

        def row_body(it, carry):
            acc = list(carry)
            git_vec = jnp.full((LANES,), it, jnp.int32)
            off = row_off + it * (UNROLL * LANES)
            for u in range(UNROLL):
                bv, bi = acc[2 * u], acc[2 * u + 1]
                v = buf[pl.ds(off + u * LANES, LANES)]
                upd = v < bv
                acc[2 * u] = jnp.where(upd, v, bv)
                acc[2 * u + 1] = jnp.where(upd, git_vec, bi)
            return tuple(acc)

        init = []
        for _ in range(UNROLL):
            init += [jnp.full((LANES,), jnp.inf, jnp.float32),
                     jnp.zeros((LANES,), jnp.int32)]
        carry = lax.fori_loop(0, ITERS_PER_ROW, row_body, tuple(init))

        mv, mi = None, None
        for u in range(UNROLL):
            gidx = carry[2 * u + 1] * (UNROLL * LANES) + (lane_iota + u * LANES)
            if mv is None:
                mv, mi = carry[2 * u], gidx
            else:
                mv, mi = _lex_min(mv, mi, carry[2 * u], gidx)

        for shift in (8, 4, 2, 1):
            partner = lane_iota ^ shift
            pv = mv.at[partner].get(mode="promise_in_bounds", unique_indices=True)
            pi = mi.at[partner].get(mode="promise_in_bounds", unique_indices=True)
            mv, mi = _lex_min(mv, mi, pv, pi)

        return jnp.where(lane_iota == r, mi, res_vec)

    res_vec = lax.fori_loop(0, ROWS_PER_WORKER, per_row,
                            jnp.zeros((LANES,), jnp.int32))
    res_v[...] = res_vec
    pltpu.sync_copy(res_v, out_hbm.at[wid])


def kernel(x):
    out2d = _argmin_sc(x)
    return out2d[:, :ROWS_PER_WORKER].reshape(-1)
